# baseline (speedup 1.0000x reference)
import functools

import jax
import jax.numpy as jnp
from jax import lax
from jax.experimental import pallas as pl
from jax.experimental.pallas import tpu as pltpu
from jax.experimental.pallas import tpu_sc as plsc

_D = 1024
_H = 16
_DH = _D // _H
_BUCKET = 16
_NH = 4
_T = 2080
_BH = 32
_NBK = _T // _BUCKET // 2 * 2
_ROWS = _BH * _NH
_CH = 80
_NCHK = _T // _CH
_HISTP = 144
_NCHUNKS = _NH * (_T // _BUCKET)
_OW = 128
_G = 13
_GR = _G * _BUCKET
_NG = (_T // _BUCKET) // _G


def _mm_t_kernel(x_ref, w_ref, o_ref):
    o_ref[...] = jax.lax.dot_general(
        x_ref[...], w_ref[...], (((1,), (1,)), ((), ())),
        preferred_element_type=jnp.float32)


def _matmul_t(x, w, block_rows=520):
    R, K = x.shape
    N = w.shape[0]
    return pl.pallas_call(
        _mm_t_kernel,
        grid=(R // block_rows,),
        in_specs=[
            pl.BlockSpec((block_rows, K), lambda i: (i, 0)),
            pl.BlockSpec((N, K), lambda i: (0, 0)),
        ],
        out_specs=pl.BlockSpec((block_rows, N), lambda i: (i, 0)),
        out_shape=jax.ShapeDtypeStruct((R, N), jnp.float32),
    )(x, w)


def _attn_kernel(sq_ref, tail_ref, str_ref, stt_ref, stc_ref, o_ref):
    x = sq_ref[0]
    tl = tail_ref[0]
    ext = jnp.concatenate([tl, x], axis=0)
    kall = ext[:, :_DH]
    nrm = jnp.sqrt(jnp.sum(kall * kall, axis=1, keepdims=True))
    kn = kall / jnp.maximum(nrm, 1e-12)
    vall = ext[:, _DH:]
    stk_all = jnp.concatenate([stt_ref[0], str_ref[0]], axis=1)
    ri = jax.lax.broadcasted_iota(jnp.int32, (_GR, _GR + _BUCKET), 0)
    ci = jax.lax.broadcasted_iota(jnp.int32, (_GR, _GR + _BUCKET), 1)
    qc = ri // _BUCKET
    cc = ci // _BUCKET
    bandneg = jnp.where((cc == qc) | (cc == qc + 1), 0.0, -1e30)
    for g in range(_NG):
        r0 = g * _GR
        q = x[r0:r0 + _GR, :_DH]
        k = kn[r0:r0 + _GR + _BUCKET]
        v = vall[r0:r0 + _GR + _BUCKET]
        d = jax.lax.dot_general(
            q, k, (((1,), (1,)), ((), ())),
            preferred_element_type=jnp.float32) * (_DH ** -0.5)
        stq = stc_ref[0][r0:r0 + _GR]
        stk = stk_all[:, r0:r0 + _GR + _BUCKET]
        d = jnp.where(stq == stk, -5e4, d) + bandneg
        m = jnp.max(d, axis=1, keepdims=True)
        ex = jnp.exp(d - m)
        s = jnp.sum(ex, axis=1, keepdims=True)
        lse = m + jnp.log(s)
        p = jnp.exp(d - lse)
        bo = jax.lax.dot_general(
            p, v, (((1,), (0,)), ((), ())),
            preferred_element_type=jnp.float32)
        o_ref[0, r0:r0 + _GR, :_DH] = bo
        o_ref[0, r0:r0 + _GR, _DH:] = jnp.broadcast_to(lse, (_GR, _OW - _DH))


def _attn(sqkv, st):
    sq = sqkv.reshape(_ROWS, _T, 2 * _DH)
    tails = sq[:, _T - _BUCKET:, :]
    st_r = st.reshape(_ROWS, 1, _T)
    stt_r = st[:, _T - _BUCKET:].reshape(_ROWS, 1, _BUCKET)
    st_c = st.reshape(_ROWS, _T, 1)
    prev = lambda b, h: b * _NH + (h + _NH - 1) % _NH
    return pl.pallas_call(
        _attn_kernel,
        grid=(_BH, _NH),
        in_specs=[
            pl.BlockSpec((1, _T, 2 * _DH), lambda b, h: (b * _NH + h, 0, 0)),
            pl.BlockSpec((1, _BUCKET, 2 * _DH), lambda b, h: (prev(b, h), 0, 0)),
            pl.BlockSpec((1, 1, _T), lambda b, h: (b * _NH + h, 0, 0)),
            pl.BlockSpec((1, 1, _BUCKET), lambda b, h: (prev(b, h), 0, 0)),
            pl.BlockSpec((1, _T, 1), lambda b, h: (b * _NH + h, 0, 0)),
        ],
        out_specs=pl.BlockSpec((1, _T, _OW), lambda b, h: (b * _NH + h, 0, 0)),
        out_shape=jax.ShapeDtypeStruct((_ROWS, _T, _OW), jnp.float32),
        compiler_params=pltpu.CompilerParams(
            dimension_semantics=("parallel", "parallel")),
    )(sq, tails, st_r, stt_r, st_c)


_sc_mesh = plsc.VectorSubcoreMesh(core_axis_name="c", subcore_axis_name="s")


@functools.partial(
    pl.kernel, mesh=_sc_mesh,
    compiler_params=pltpu.CompilerParams(needs_layout_passes=False),
    out_type=[
        jax.ShapeDtypeStruct((_ROWS, _T), jnp.int32),
        jax.ShapeDtypeStruct((_ROWS, _NCHK, _CH), jnp.int32),
        jax.ShapeDtypeStruct((_BH * _NH * _T, 2 * _DH), jnp.float32),
    ],
    scratch_types=[
        pltpu.VMEM((_T,), jnp.int32),
        pltpu.VMEM((_T,), jnp.int32),
        pltpu.VMEM((_HISTP,), jnp.int32),
        pltpu.VMEM((_NCHK, _CH), jnp.int32),
        pltpu.VMEM((_NCHK, _CH), jnp.int32),
        pltpu.VMEM((_CH, 2 * _DH), jnp.float32),
        pltpu.SemaphoreType.DMA,
    ])
def _sc_sort_gather(bkt_hbm, qkv_hbm, st_hbm, gidx2_hbm, sqkv_hbm,
                    bkt_v, st_v, hist_v, sidx_v, g2_v, buf_v, sem):
    wid = lax.axis_index("s") * 2 + lax.axis_index("c")

    def row_body(j, carry):
        r = wid * 4 + j
        bh = r // _NH
        qk_base = bh * _T
        out_base = r * _T

        pltpu.sync_copy(bkt_hbm.at[r], bkt_v)

        def z_body(i, c):
            hist_v[pl.ds(i * 16, 16)] = jnp.zeros((16,), jnp.int32)
            return c
        lax.fori_loop(0, _HISTP // 16, z_body, 0)

        def h_body(i, c):
            b = bkt_v[pl.ds(i * 16, 16)]
            cnt, last = plsc.scan_count(b)
            plsc.addupdate_scatter(hist_v, [b], cnt, mask=last)
            return c
        lax.fori_loop(0, _T // 16, h_body, 0)

        def s_body(i, c):
            v = hist_v[pl.ds(i * 16, 16)]
            inc = plsc.cumsum(v)
            hist_v[pl.ds(i * 16, 16)] = inc - v + c
            return c + jnp.sum(v)
        lax.fori_loop(0, _HISTP // 16, s_body, 0)

        def p_body(i, c):
            b = bkt_v[pl.ds(i * 16, 16)]
            cnt, last = plsc.scan_count(b)
            cur = plsc.load_gather(hist_v, [b])
            p = cur + cnt - 1
            t = lax.iota(jnp.int32, 16) + i * 16
            plsc.store_scatter(st_v, [p], t)
            plsc.store_scatter(sidx_v, [p // _CH, p % _CH], t + qk_base)
            plsc.store_scatter(g2_v, [t // _CH, t % _CH], p + out_base)
            plsc.addupdate_scatter(hist_v, [b], cnt, mask=last)
            return c
        lax.fori_loop(0, _T // 16, p_body, 0)

        pltpu.sync_copy(st_v, st_hbm.at[r])
        pltpu.sync_copy(g2_v, gidx2_hbm.at[r])

        def c_body(k, c):
            pltpu.async_copy(qkv_hbm.at[sidx_v.at[k]], buf_v, sem).wait()
            pltpu.sync_copy(buf_v, sqkv_hbm.at[pl.ds(out_base + k * _CH, _CH)])
            return c
        lax.fori_loop(0, _NCHK, c_body, 0)
        return carry

    lax.fori_loop(0, _ROWS // 32, row_body, 0)


@functools.partial(
    pl.kernel, mesh=_sc_mesh,
    compiler_params=pltpu.CompilerParams(needs_layout_passes=False),
    out_type=jax.ShapeDtypeStruct((_BH * _NH * _T, _OW), jnp.float32),
    scratch_types=[
        pltpu.VMEM((_NCHK, _CH), jnp.int32),
        pltpu.VMEM((_CH, _OW), jnp.float32),
        pltpu.SemaphoreType.DMA,
    ])
def _sc_unsort(sol_hbm, gidx2_hbm, ou_hbm, g2_v, buf_v, sem):
    wid = lax.axis_index("s") * 2 + lax.axis_index("c")

    def row_body(j, carry):
        r = wid * 4 + j
        out_base = r * _T
        pltpu.sync_copy(gidx2_hbm.at[r], g2_v)

        def c_body(k, c):
            pltpu.async_copy(sol_hbm.at[g2_v.at[k]], buf_v, sem).wait()
            pltpu.sync_copy(buf_v, ou_hbm.at[pl.ds(out_base + k * _CH, _CH)])
            return c
        lax.fori_loop(0, _NCHK, c_body, 0)
        return carry

    lax.fori_loop(0, _ROWS // 32, row_body, 0)


def _buckets(qkh):
    rot = jax.random.normal(jax.random.key(42), (_DH, _NH, _NBK // 2),
                            dtype=jnp.float32)
    rotated = jnp.einsum('btf,fhi->bhti', qkh, rot)
    rotated = jnp.concatenate([rotated, -rotated], axis=-1)
    return jnp.argmax(rotated, axis=-1).astype(jnp.int32)


def kernel(x, W_qk, W_v, W_out, b_out):
    b, l, d = x.shape
    pad = 2 * _BUCKET - l % (2 * _BUCKET)
    xp = jnp.concatenate([x, jnp.zeros((b, pad, d), x.dtype)], axis=1)
    t = l + pad
    xf = xp.reshape(b * t, d)
    qk = _matmul_t(xf, W_qk).reshape(b, t, d)
    v = _matmul_t(xf, W_v).reshape(b, t, d)

    def split_heads(a):
        return a.reshape(b, t, _H, _DH).transpose(0, 2, 1, 3).reshape(b * _H, t, _DH)

    qkh = split_heads(qk)
    vh = split_heads(v)

    bkt = _buckets(qkh).reshape(_ROWS, _T)
    qkv = jnp.concatenate([qkh, vh], axis=-1).reshape(_BH * _T, 2 * _DH)

    st, gidx2, sqkv = _sc_sort_gather(bkt, qkv)

    sol = _attn(sqkv, st).reshape(_BH * _NH * _T, _OW)
    ou = _sc_unsort(sol, gidx2).reshape(_BH, _NH, _T, _OW)

    o = ou[..., :_DH]
    logits = ou[..., _DH]
    probs = jnp.exp(logits - jax.nn.logsumexp(logits, axis=1, keepdims=True))
    ctx = jnp.sum(o * probs[..., None], axis=1)

    attn = ctx.reshape(b, _H, t, _DH).transpose(0, 2, 1, 3).reshape(b, t, d)
    out = _matmul_t(attn.reshape(b * t, d), W_out).reshape(b, t, d) + b_out
    return out[:, :-pad, :]


# speedup vs baseline: 1.0729x; 1.0729x over previous
import functools

import jax
import jax.numpy as jnp
from jax import lax
from jax.experimental import pallas as pl
from jax.experimental.pallas import tpu as pltpu
from jax.experimental.pallas import tpu_sc as plsc

_D = 1024
_H = 16
_DH = _D // _H
_BUCKET = 16
_NH = 4
_T = 2080
_BH = 32
_NBK = _T // _BUCKET // 2 * 2
_ROWS = _BH * _NH
_CH = 80
_NCHK = _T // _CH
_HISTP = 144
_NCHUNKS = _NH * (_T // _BUCKET)
_OW = 128
_G = 13
_GR = _G * _BUCKET
_NG = (_T // _BUCKET) // _G


def _mm_t_kernel(x_ref, w_ref, o_ref):
    o_ref[...] = jax.lax.dot_general(
        x_ref[...], w_ref[...], (((1,), (1,)), ((), ())),
        preferred_element_type=jnp.float32)


def _matmul_t(x, w, block_rows=520):
    R, K = x.shape
    N = w.shape[0]
    return pl.pallas_call(
        _mm_t_kernel,
        grid=(R // block_rows,),
        in_specs=[
            pl.BlockSpec((block_rows, K), lambda i: (i, 0)),
            pl.BlockSpec((N, K), lambda i: (0, 0)),
        ],
        out_specs=pl.BlockSpec((block_rows, N), lambda i: (i, 0)),
        out_shape=jax.ShapeDtypeStruct((R, N), jnp.float32),
    )(x, w)


def _qkv_kernel(x_ref, wqk_ref, wv_ref, o_ref):
    x = x_ref[0]
    qk = jax.lax.dot_general(
        x, wqk_ref[...], (((1,), (1,)), ((), ())),
        preferred_element_type=jnp.float32)
    v = jax.lax.dot_general(
        x, wv_ref[...], (((1,), (1,)), ((), ())),
        preferred_element_type=jnp.float32)
    o_ref[0, :, :_DH] = qk
    o_ref[0, :, _DH:] = v


def _qkv(xp, W_qk, W_v):
    b = xp.shape[0]
    return pl.pallas_call(
        _qkv_kernel,
        grid=(b, _H),
        in_specs=[
            pl.BlockSpec((1, _T, _D), lambda i, h: (i, 0, 0)),
            pl.BlockSpec((_DH, _D), lambda i, h: (h, 0)),
            pl.BlockSpec((_DH, _D), lambda i, h: (h, 0)),
        ],
        out_specs=pl.BlockSpec((1, _T, 2 * _DH), lambda i, h: (i * _H + h, 0, 0)),
        out_shape=jax.ShapeDtypeStruct((_BH, _T, 2 * _DH), jnp.float32),
    )(xp, W_qk, W_v)


def _bucket_kernel(qkv_ref, rot_ref, b_ref):
    x = qkv_ref[0]
    rT = jax.lax.dot_general(
        rot_ref[...], x[:, :_DH], (((0,), (1,)), ((), ())),
        preferred_element_type=jnp.float32)
    nb2 = _NBK // 2
    si = jax.lax.broadcasted_iota(jnp.int32, (nb2, _T), 0)
    for h in range(_NH):
        rh = rT[h * nb2:(h + 1) * nb2, :]
        mp = jnp.max(rh, axis=0, keepdims=True)
        mn = jnp.max(-rh, axis=0, keepdims=True)
        ap = jnp.min(jnp.where(rh == mp, si, nb2), axis=0, keepdims=True)
        an = jnp.min(jnp.where(-rh == mn, si, nb2), axis=0, keepdims=True)
        b_ref[h:h + 1, :] = jnp.where(mp >= mn, ap, nb2 + an)


def _bucketize(qkv, rot):
    return pl.pallas_call(
        _bucket_kernel,
        grid=(_BH,),
        in_specs=[
            pl.BlockSpec((1, _T, 2 * _DH), lambda i: (i, 0, 0)),
            pl.BlockSpec((_DH, _NH * (_NBK // 2)), lambda i: (0, 0)),
        ],
        out_specs=pl.BlockSpec((_NH, _T), lambda i: (i, 0)),
        out_shape=jax.ShapeDtypeStruct((_ROWS, _T), jnp.int32),
    )(qkv, rot)


def _combine_kernel(ou_ref, o_ref):
    l = jnp.concatenate(
        [ou_ref[0, h, :, _DH:_DH + 1] for h in range(_NH)], axis=1)
    m = jnp.max(l, axis=1, keepdims=True)
    e = jnp.exp(l - m)
    s = jnp.sum(e, axis=1, keepdims=True)
    w = jnp.exp(l - m - jnp.log(s))
    acc = ou_ref[0, 0, :, :_DH] * w[:, 0:1]
    for h in range(1, _NH):
        acc = acc + ou_ref[0, h, :, :_DH] * w[:, h:h + 1]
    o_ref[0, :, 0, :] = acc


def _combine(ou4):
    return pl.pallas_call(
        _combine_kernel,
        grid=(_BH,),
        in_specs=[pl.BlockSpec((1, _NH, _T, _OW), lambda i: (i, 0, 0, 0))],
        out_specs=pl.BlockSpec((1, _T, 1, _DH), lambda i: (i // _H, 0, i % _H, 0)),
        out_shape=jax.ShapeDtypeStruct((_BH // _H, _T, _H, _DH), jnp.float32),
    )(ou4)


def _attn_kernel(sq_ref, tail_ref, str_ref, stt_ref, stc_ref, o_ref):
    x = sq_ref[0]
    tl = tail_ref[0]
    ext = jnp.concatenate([tl, x], axis=0)
    kall = ext[:, :_DH]
    nrm = jnp.sqrt(jnp.sum(kall * kall, axis=1, keepdims=True))
    kn = kall / jnp.maximum(nrm, 1e-12)
    vall = ext[:, _DH:]
    stk_all = jnp.concatenate([stt_ref[0], str_ref[0]], axis=1)
    ri = jax.lax.broadcasted_iota(jnp.int32, (_GR, _GR + _BUCKET), 0)
    ci = jax.lax.broadcasted_iota(jnp.int32, (_GR, _GR + _BUCKET), 1)
    qc = ri // _BUCKET
    cc = ci // _BUCKET
    bandneg = jnp.where((cc == qc) | (cc == qc + 1), 0.0, -1e30)
    for g in range(_NG):
        r0 = g * _GR
        q = x[r0:r0 + _GR, :_DH]
        k = kn[r0:r0 + _GR + _BUCKET]
        v = vall[r0:r0 + _GR + _BUCKET]
        d = jax.lax.dot_general(
            q, k, (((1,), (1,)), ((), ())),
            preferred_element_type=jnp.float32) * (_DH ** -0.5)
        stq = stc_ref[0][r0:r0 + _GR]
        stk = stk_all[:, r0:r0 + _GR + _BUCKET]
        d = jnp.where(stq == stk, -5e4, d) + bandneg
        m = jnp.max(d, axis=1, keepdims=True)
        ex = jnp.exp(d - m)
        s = jnp.sum(ex, axis=1, keepdims=True)
        lse = m + jnp.log(s)
        r = 1.0 / s
        r = r * (2.0 - s * r)
        bo = jax.lax.dot_general(
            ex, v, (((1,), (0,)), ((), ())),
            preferred_element_type=jnp.float32) * r
        o_ref[0, r0:r0 + _GR, :_DH] = bo
        o_ref[0, r0:r0 + _GR, _DH:] = jnp.broadcast_to(lse, (_GR, _OW - _DH))


def _attn(sqkv, st):
    sq = sqkv.reshape(_ROWS, _T, 2 * _DH)
    tails = sq[:, _T - _BUCKET:, :]
    st_r = st.reshape(_ROWS, 1, _T)
    stt_r = st[:, _T - _BUCKET:].reshape(_ROWS, 1, _BUCKET)
    st_c = st.reshape(_ROWS, _T, 1)
    prev = lambda b, h: b * _NH + (h + _NH - 1) % _NH
    return pl.pallas_call(
        _attn_kernel,
        grid=(_BH, _NH),
        in_specs=[
            pl.BlockSpec((1, _T, 2 * _DH), lambda b, h: (b * _NH + h, 0, 0)),
            pl.BlockSpec((1, _BUCKET, 2 * _DH), lambda b, h: (prev(b, h), 0, 0)),
            pl.BlockSpec((1, 1, _T), lambda b, h: (b * _NH + h, 0, 0)),
            pl.BlockSpec((1, 1, _BUCKET), lambda b, h: (prev(b, h), 0, 0)),
            pl.BlockSpec((1, _T, 1), lambda b, h: (b * _NH + h, 0, 0)),
        ],
        out_specs=pl.BlockSpec((1, _T, _OW), lambda b, h: (b * _NH + h, 0, 0)),
        out_shape=jax.ShapeDtypeStruct((_ROWS, _T, _OW), jnp.float32),
        compiler_params=pltpu.CompilerParams(
            dimension_semantics=("parallel", "parallel")),
    )(sq, tails, st_r, stt_r, st_c)


_sc_mesh = plsc.VectorSubcoreMesh(core_axis_name="c", subcore_axis_name="s")


@functools.partial(
    pl.kernel, mesh=_sc_mesh,
    compiler_params=pltpu.CompilerParams(needs_layout_passes=False),
    out_type=[
        jax.ShapeDtypeStruct((_ROWS, _T), jnp.int32),
        jax.ShapeDtypeStruct((_ROWS, _NCHK, _CH), jnp.int32),
        jax.ShapeDtypeStruct((_BH * _NH * _T, 2 * _DH), jnp.float32),
    ],
    scratch_types=[
        pltpu.VMEM((_T,), jnp.int32),
        pltpu.VMEM((_T,), jnp.int32),
        pltpu.VMEM((_HISTP,), jnp.int32),
        pltpu.VMEM((_NCHK, _CH), jnp.int32),
        pltpu.VMEM((_NCHK, _CH), jnp.int32),
        pltpu.VMEM((_CH, 2 * _DH), jnp.float32),
        pltpu.SemaphoreType.DMA,
    ])
def _sc_sort_gather(bkt_hbm, qkv_hbm, st_hbm, gidx2_hbm, sqkv_hbm,
                    bkt_v, st_v, hist_v, sidx_v, g2_v, buf_v, sem):
    wid = lax.axis_index("s") * 2 + lax.axis_index("c")

    def row_body(j, carry):
        r = wid * 4 + j
        bh = r // _NH
        qk_base = bh * _T
        out_base = r * _T

        pltpu.sync_copy(bkt_hbm.at[r], bkt_v)

        def z_body(i, c):
            hist_v[pl.ds(i * 16, 16)] = jnp.zeros((16,), jnp.int32)
            return c
        lax.fori_loop(0, _HISTP // 16, z_body, 0)

        def h_body(i, c):
            b = bkt_v[pl.ds(i * 16, 16)]
            cnt, last = plsc.scan_count(b)
            plsc.addupdate_scatter(hist_v, [b], cnt, mask=last)
            return c
        lax.fori_loop(0, _T // 16, h_body, 0)

        def s_body(i, c):
            v = hist_v[pl.ds(i * 16, 16)]
            inc = plsc.cumsum(v)
            hist_v[pl.ds(i * 16, 16)] = inc - v + c
            return c + jnp.sum(v)
        lax.fori_loop(0, _HISTP // 16, s_body, 0)

        def p_body(i, c):
            b = bkt_v[pl.ds(i * 16, 16)]
            cnt, last = plsc.scan_count(b)
            cur = plsc.load_gather(hist_v, [b])
            p = cur + cnt - 1
            t = lax.iota(jnp.int32, 16) + i * 16
            plsc.store_scatter(st_v, [p], t)
            plsc.store_scatter(sidx_v, [p // _CH, p % _CH], t + qk_base)
            plsc.store_scatter(g2_v, [t // _CH, t % _CH], p + out_base)
            plsc.addupdate_scatter(hist_v, [b], cnt, mask=last)
            return c
        lax.fori_loop(0, _T // 16, p_body, 0)

        pltpu.sync_copy(st_v, st_hbm.at[r])
        pltpu.sync_copy(g2_v, gidx2_hbm.at[r])

        def c_body(k, c):
            pltpu.async_copy(qkv_hbm.at[sidx_v.at[k]], buf_v, sem).wait()
            pltpu.sync_copy(buf_v, sqkv_hbm.at[pl.ds(out_base + k * _CH, _CH)])
            return c
        lax.fori_loop(0, _NCHK, c_body, 0)
        return carry

    lax.fori_loop(0, _ROWS // 32, row_body, 0)


@functools.partial(
    pl.kernel, mesh=_sc_mesh,
    compiler_params=pltpu.CompilerParams(needs_layout_passes=False),
    out_type=jax.ShapeDtypeStruct((_BH * _NH * _T, _OW), jnp.float32),
    scratch_types=[
        pltpu.VMEM((_NCHK, _CH), jnp.int32),
        pltpu.VMEM((_CH, _OW), jnp.float32),
        pltpu.SemaphoreType.DMA,
    ])
def _sc_unsort(sol_hbm, gidx2_hbm, ou_hbm, g2_v, buf_v, sem):
    wid = lax.axis_index("s") * 2 + lax.axis_index("c")

    def row_body(j, carry):
        r = wid * 4 + j
        out_base = r * _T
        pltpu.sync_copy(gidx2_hbm.at[r], g2_v)

        def c_body(k, c):
            pltpu.async_copy(sol_hbm.at[g2_v.at[k]], buf_v, sem).wait()
            pltpu.sync_copy(buf_v, ou_hbm.at[pl.ds(out_base + k * _CH, _CH)])
            return c
        lax.fori_loop(0, _NCHK, c_body, 0)
        return carry

    lax.fori_loop(0, _ROWS // 32, row_body, 0)


def _buckets(qkh):
    rot = jax.random.normal(jax.random.key(42), (_DH, _NH, _NBK // 2),
                            dtype=jnp.float32)
    rotated = jnp.einsum('btf,fhi->bhti', qkh, rot)
    rotated = jnp.concatenate([rotated, -rotated], axis=-1)
    return jnp.argmax(rotated, axis=-1).astype(jnp.int32)


def kernel(x, W_qk, W_v, W_out, b_out):
    b, l, d = x.shape
    pad = 2 * _BUCKET - l % (2 * _BUCKET)
    xp = jnp.concatenate([x, jnp.zeros((b, pad, d), x.dtype)], axis=1)
    t = l + pad
    xf = xp.reshape(b * t, d)
    qk = _matmul_t(xf, W_qk).reshape(b, t, d)
    v = _matmul_t(xf, W_v).reshape(b, t, d)

    def split_heads(a):
        return a.reshape(b, t, _H, _DH).transpose(0, 2, 1, 3).reshape(b * _H, t, _DH)

    qkh = split_heads(qk)
    vh = split_heads(v)

    bkt = _buckets(qkh).reshape(_ROWS, _T)
    qkv = jnp.concatenate([qkh, vh], axis=-1).reshape(_BH * _T, 2 * _DH)

    st, gidx2, sqkv = _sc_sort_gather(bkt, qkv)

    sol = _attn(sqkv, st).reshape(_BH * _NH * _T, _OW)
    ou = _sc_unsort(sol, gidx2).reshape(_BH, _NH, _T, _OW)

    o = ou[..., :_DH]
    logits = ou[..., _DH]
    probs = jnp.exp(logits - jax.nn.logsumexp(logits, axis=1, keepdims=True))
    ctx = jnp.sum(o * probs[..., None], axis=1)

    attn = ctx.reshape(b, _H, t, _DH).transpose(0, 2, 1, 3).reshape(b, t, d)
    out = _matmul_t(attn.reshape(b * t, d), W_out).reshape(b, t, d) + b_out
    return out[:, :-pad, :]


# speedup vs baseline: 1.3202x; 1.2306x over previous
import functools

import jax
import jax.numpy as jnp
from jax import lax
from jax.experimental import pallas as pl
from jax.experimental.pallas import tpu as pltpu
from jax.experimental.pallas import tpu_sc as plsc

_D = 1024
_H = 16
_DH = _D // _H
_BUCKET = 16
_NH = 4
_T = 2080
_BH = 32
_NBK = _T // _BUCKET // 2 * 2
_ROWS = _BH * _NH
_CH = 80
_NCHK = _T // _CH
_HISTP = 144
_NCHUNKS = _NH * (_T // _BUCKET)
_OW = 128
_G = 13
_GR = _G * _BUCKET
_NG = (_T // _BUCKET) // _G


def _mm_t_kernel(x_ref, w_ref, o_ref):
    o_ref[...] = jax.lax.dot_general(
        x_ref[...], w_ref[...], (((1,), (1,)), ((), ())),
        preferred_element_type=jnp.float32)


def _matmul_t(x, w, block_rows=520):
    R, K = x.shape
    N = w.shape[0]
    return pl.pallas_call(
        _mm_t_kernel,
        grid=(R // block_rows,),
        in_specs=[
            pl.BlockSpec((block_rows, K), lambda i: (i, 0)),
            pl.BlockSpec((N, K), lambda i: (0, 0)),
        ],
        out_specs=pl.BlockSpec((block_rows, N), lambda i: (i, 0)),
        out_shape=jax.ShapeDtypeStruct((R, N), jnp.float32),
    )(x, w)


def _qkv_kernel(x_ref, wqk_ref, wv_ref, o_ref):
    x = x_ref[0]
    qk = jax.lax.dot_general(
        x, wqk_ref[...], (((1,), (1,)), ((), ())),
        preferred_element_type=jnp.float32)
    v = jax.lax.dot_general(
        x, wv_ref[...], (((1,), (1,)), ((), ())),
        preferred_element_type=jnp.float32)
    o_ref[0, :, :_DH] = qk
    o_ref[0, :, _DH:] = v


def _qkv(xp, W_qk, W_v):
    b = xp.shape[0]
    return pl.pallas_call(
        _qkv_kernel,
        grid=(b, _H),
        in_specs=[
            pl.BlockSpec((1, _T, _D), lambda i, h: (i, 0, 0)),
            pl.BlockSpec((_DH, _D), lambda i, h: (h, 0)),
            pl.BlockSpec((_DH, _D), lambda i, h: (h, 0)),
        ],
        out_specs=pl.BlockSpec((1, _T, 2 * _DH), lambda i, h: (i * _H + h, 0, 0)),
        out_shape=jax.ShapeDtypeStruct((_BH, _T, 2 * _DH), jnp.float32),
    )(xp, W_qk, W_v)


def _bucket_kernel(qkv_ref, rot_ref, b_ref):
    x = qkv_ref[0]
    rT = jax.lax.dot_general(
        rot_ref[...], x[:, :_DH], (((0,), (1,)), ((), ())),
        preferred_element_type=jnp.float32)
    nb2 = _NBK // 2
    si = jax.lax.broadcasted_iota(jnp.int32, (nb2, _T), 0)
    for h in range(_NH):
        rh = rT[h * nb2:(h + 1) * nb2, :]
        mp = jnp.max(rh, axis=0, keepdims=True)
        mn = jnp.max(-rh, axis=0, keepdims=True)
        ap = jnp.min(jnp.where(rh == mp, si, nb2), axis=0, keepdims=True)
        an = jnp.min(jnp.where(-rh == mn, si, nb2), axis=0, keepdims=True)
        b_ref[0, h:h + 1, :] = jnp.where(mp >= mn, ap, nb2 + an)


def _bucketize(qkv, rot):
    return pl.pallas_call(
        _bucket_kernel,
        grid=(_BH,),
        in_specs=[
            pl.BlockSpec((1, _T, 2 * _DH), lambda i: (i, 0, 0)),
            pl.BlockSpec((_DH, _NH * (_NBK // 2)), lambda i: (0, 0)),
        ],
        out_specs=pl.BlockSpec((1, _NH, _T), lambda i: (i, 0, 0)),
        out_shape=jax.ShapeDtypeStruct((_BH, _NH, _T), jnp.int32),
    )(qkv, rot)


def _combine_kernel(ou_ref, o_ref):
    l = jnp.concatenate(
        [ou_ref[0, h, :, _DH:_DH + 1] for h in range(_NH)], axis=1)
    m = jnp.max(l, axis=1, keepdims=True)
    e = jnp.exp(l - m)
    s = jnp.sum(e, axis=1, keepdims=True)
    w = jnp.exp(l - m - jnp.log(s))
    acc = ou_ref[0, 0, :, :_DH] * w[:, 0:1]
    for h in range(1, _NH):
        acc = acc + ou_ref[0, h, :, :_DH] * w[:, h:h + 1]
    o_ref[0] = acc


def _combine(ou4):
    return pl.pallas_call(
        _combine_kernel,
        grid=(_BH,),
        in_specs=[pl.BlockSpec((1, _NH, _T, _OW), lambda i: (i, 0, 0, 0))],
        out_specs=pl.BlockSpec((1, _T, _DH), lambda i: (i, 0, 0)),
        out_shape=jax.ShapeDtypeStruct((_BH, _T, _DH), jnp.float32),
    )(ou4)


def _attn_kernel(sq_ref, tail_ref, str_ref, stt_ref, stc_ref, o_ref):
    x = sq_ref[0]
    tl = tail_ref[0]
    ext = jnp.concatenate([tl, x], axis=0)
    kall = ext[:, :_DH]
    nrm = jnp.sqrt(jnp.sum(kall * kall, axis=1, keepdims=True))
    kn = kall / jnp.maximum(nrm, 1e-12)
    vall = ext[:, _DH:]
    stk_all = jnp.concatenate([stt_ref[0], str_ref[0]], axis=1)
    ri = jax.lax.broadcasted_iota(jnp.int32, (_GR, _GR + _BUCKET), 0)
    ci = jax.lax.broadcasted_iota(jnp.int32, (_GR, _GR + _BUCKET), 1)
    qc = ri // _BUCKET
    cc = ci // _BUCKET
    bandneg = jnp.where((cc == qc) | (cc == qc + 1), 0.0, -1e30)
    for g in range(_NG):
        r0 = g * _GR
        q = x[r0:r0 + _GR, :_DH]
        k = kn[r0:r0 + _GR + _BUCKET]
        v = vall[r0:r0 + _GR + _BUCKET]
        d = jax.lax.dot_general(
            q, k, (((1,), (1,)), ((), ())),
            preferred_element_type=jnp.float32) * (_DH ** -0.5)
        stq = stc_ref[0][r0:r0 + _GR]
        stk = stk_all[:, r0:r0 + _GR + _BUCKET]
        d = jnp.where(stq == stk, -5e4, d) + bandneg
        m = jnp.max(d, axis=1, keepdims=True)
        ex = jnp.exp(d - m)
        s = jnp.sum(ex, axis=1, keepdims=True)
        lse = m + jnp.log(s)
        r = 1.0 / s
        r = r * (2.0 - s * r)
        bo = jax.lax.dot_general(
            ex, v, (((1,), (0,)), ((), ())),
            preferred_element_type=jnp.float32) * r
        o_ref[0, r0:r0 + _GR, :_DH] = bo
        o_ref[0, r0:r0 + _GR, _DH:] = jnp.broadcast_to(lse, (_GR, _OW - _DH))


def _attn(sqkv, st):
    sq = sqkv.reshape(_ROWS, _T, 2 * _DH)
    tails = sq[:, _T - _BUCKET:, :]
    st_r = st.reshape(_ROWS, 1, _T)
    stt_r = st[:, _T - _BUCKET:].reshape(_ROWS, 1, _BUCKET)
    st_c = st.reshape(_ROWS, _T, 1)
    prev = lambda b, h: b * _NH + (h + _NH - 1) % _NH
    return pl.pallas_call(
        _attn_kernel,
        grid=(_BH, _NH),
        in_specs=[
            pl.BlockSpec((1, _T, 2 * _DH), lambda b, h: (b * _NH + h, 0, 0)),
            pl.BlockSpec((1, _BUCKET, 2 * _DH), lambda b, h: (prev(b, h), 0, 0)),
            pl.BlockSpec((1, 1, _T), lambda b, h: (b * _NH + h, 0, 0)),
            pl.BlockSpec((1, 1, _BUCKET), lambda b, h: (prev(b, h), 0, 0)),
            pl.BlockSpec((1, _T, 1), lambda b, h: (b * _NH + h, 0, 0)),
        ],
        out_specs=pl.BlockSpec((1, _T, _OW), lambda b, h: (b * _NH + h, 0, 0)),
        out_shape=jax.ShapeDtypeStruct((_ROWS, _T, _OW), jnp.float32),
        compiler_params=pltpu.CompilerParams(
            dimension_semantics=("parallel", "parallel")),
    )(sq, tails, st_r, stt_r, st_c)


_sc_mesh = plsc.VectorSubcoreMesh(core_axis_name="c", subcore_axis_name="s")


@functools.partial(
    pl.kernel, mesh=_sc_mesh,
    compiler_params=pltpu.CompilerParams(needs_layout_passes=False),
    out_type=[
        jax.ShapeDtypeStruct((_ROWS, _T), jnp.int32),
        jax.ShapeDtypeStruct((_ROWS, _NCHK, _CH), jnp.int32),
        jax.ShapeDtypeStruct((_BH * _NH * _T, 2 * _DH), jnp.float32),
    ],
    scratch_types=[
        pltpu.VMEM((_T,), jnp.int32),
        pltpu.VMEM((_T,), jnp.int32),
        pltpu.VMEM((_HISTP,), jnp.int32),
        pltpu.VMEM((_NCHK, _CH), jnp.int32),
        pltpu.VMEM((_NCHK, _CH), jnp.int32),
        pltpu.VMEM((_CH, 2 * _DH), jnp.float32),
        pltpu.SemaphoreType.DMA,
    ])
def _sc_sort_gather(bkt_hbm, qkv_hbm, st_hbm, gidx2_hbm, sqkv_hbm,
                    bkt_v, st_v, hist_v, sidx_v, g2_v, buf_v, sem):
    wid = lax.axis_index("s") * 2 + lax.axis_index("c")

    def row_body(j, carry):
        r = wid * 4 + j
        bh = r // _NH
        qk_base = bh * _T
        out_base = r * _T

        pltpu.sync_copy(bkt_hbm.at[r], bkt_v)

        def z_body(i, c):
            hist_v[pl.ds(i * 16, 16)] = jnp.zeros((16,), jnp.int32)
            return c
        lax.fori_loop(0, _HISTP // 16, z_body, 0)

        def h_body(i, c):
            b = bkt_v[pl.ds(i * 16, 16)]
            cnt, last = plsc.scan_count(b)
            plsc.addupdate_scatter(hist_v, [b], cnt, mask=last)
            return c
        lax.fori_loop(0, _T // 16, h_body, 0)

        def s_body(i, c):
            v = hist_v[pl.ds(i * 16, 16)]
            inc = plsc.cumsum(v)
            hist_v[pl.ds(i * 16, 16)] = inc - v + c
            return c + jnp.sum(v)
        lax.fori_loop(0, _HISTP // 16, s_body, 0)

        def p_body(i, c):
            b = bkt_v[pl.ds(i * 16, 16)]
            cnt, last = plsc.scan_count(b)
            cur = plsc.load_gather(hist_v, [b])
            p = cur + cnt - 1
            t = lax.iota(jnp.int32, 16) + i * 16
            plsc.store_scatter(st_v, [p], t)
            plsc.store_scatter(sidx_v, [p // _CH, p % _CH], t + qk_base)
            plsc.store_scatter(g2_v, [t // _CH, t % _CH], p + out_base)
            plsc.addupdate_scatter(hist_v, [b], cnt, mask=last)
            return c
        lax.fori_loop(0, _T // 16, p_body, 0)

        pltpu.sync_copy(st_v, st_hbm.at[r])
        pltpu.sync_copy(g2_v, gidx2_hbm.at[r])

        def c_body(k, c):
            pltpu.async_copy(qkv_hbm.at[sidx_v.at[k]], buf_v, sem).wait()
            pltpu.sync_copy(buf_v, sqkv_hbm.at[pl.ds(out_base + k * _CH, _CH)])
            return c
        lax.fori_loop(0, _NCHK, c_body, 0)
        return carry

    lax.fori_loop(0, _ROWS // 32, row_body, 0)


@functools.partial(
    pl.kernel, mesh=_sc_mesh,
    compiler_params=pltpu.CompilerParams(needs_layout_passes=False),
    out_type=jax.ShapeDtypeStruct((_BH * _NH * _T, _OW), jnp.float32),
    scratch_types=[
        pltpu.VMEM((_NCHK, _CH), jnp.int32),
        pltpu.VMEM((_CH, _OW), jnp.float32),
        pltpu.SemaphoreType.DMA,
    ])
def _sc_unsort(sol_hbm, gidx2_hbm, ou_hbm, g2_v, buf_v, sem):
    wid = lax.axis_index("s") * 2 + lax.axis_index("c")

    def row_body(j, carry):
        r = wid * 4 + j
        out_base = r * _T
        pltpu.sync_copy(gidx2_hbm.at[r], g2_v)

        def c_body(k, c):
            pltpu.async_copy(sol_hbm.at[g2_v.at[k]], buf_v, sem).wait()
            pltpu.sync_copy(buf_v, ou_hbm.at[pl.ds(out_base + k * _CH, _CH)])
            return c
        lax.fori_loop(0, _NCHK, c_body, 0)
        return carry

    lax.fori_loop(0, _ROWS // 32, row_body, 0)


def kernel(x, W_qk, W_v, W_out, b_out):
    b, l, d = x.shape
    pad = 2 * _BUCKET - l % (2 * _BUCKET)
    xp = jnp.concatenate([x, jnp.zeros((b, pad, d), x.dtype)], axis=1)
    t = l + pad

    qkv3 = _qkv(xp, W_qk, W_v)
    rot = jax.random.normal(jax.random.key(42), (_DH, _NH, _NBK // 2),
                            dtype=jnp.float32).reshape(_DH, _NH * (_NBK // 2))
    bkt = _bucketize(qkv3, rot).reshape(_ROWS, _T)

    st, gidx2, sqkv = _sc_sort_gather(bkt, qkv3.reshape(_BH * _T, 2 * _DH))

    sol = _attn(sqkv, st).reshape(_BH * _NH * _T, _OW)
    ou = _sc_unsort(sol, gidx2).reshape(_BH, _NH, _T, _OW)

    ctx = _combine(ou)
    attn = ctx.reshape(b, _H, t, _DH).transpose(0, 2, 1, 3).reshape(b, t, d)
    out = _matmul_t(attn.reshape(b * t, d), W_out).reshape(b, t, d) + b_out
    return out[:, :-pad, :]
